# baseline (speedup 1.0000x reference)
import functools

import jax
import jax.numpy as jnp
import numpy as np
from jax import lax
from jax.experimental import pallas as pl
from jax.experimental.pallas import tpu as pltpu
from jax.experimental.pallas import tpu_sc as plsc

_N = 32768
_NB = 2048
_NW = 32
_ROWS_PER_W = 8 * _N // _NW

_INTERPRET = False


def _prep_body(c_ref, idx0_ref, w0_ref, idx1_ref, w1_ref):
    x = c_ref[0:1, :]
    y = c_ref[1:2, :]
    z = c_ref[2:3, :]
    p = lax.broadcasted_iota(jnp.int32, (1, _N), 1)
    b = lax.shift_right_logical(p, 13)
    for scale, cap, d, idx_ref, w_ref in (
        (np.float32(64.0), np.float32(64.0) - np.float32(1.01), 64, idx0_ref, w0_ref),
        (np.float32(32.0), np.float32(32.0) - np.float32(1.01), 32, idx1_ref, w1_ref),
    ):
        lo = []
        hi = []
        wlo = []
        whi = []
        for a in (x, y, z):
            t = jnp.clip(a * scale, np.float32(0.01), cap)
            i1 = t.astype(jnp.int32)
            f1 = i1.astype(jnp.float32)
            w_h = t - f1
            i2 = jnp.where(w_h == 0.0, i1, i1 + 1)
            w_l = i2.astype(jnp.float32) - t
            lo.append(i1); hi.append(i2); wlo.append(w_l); whi.append(w_h)
        idx_rows, w_rows = [], []
        for dx in (0, 1):
            ix = hi[0] if dx else lo[0]
            wx = whi[0] if dx else wlo[0]
            for dy in (0, 1):
                iy = hi[1] if dy else lo[1]
                wy = whi[1] if dy else wlo[1]
                for dz in (0, 1):
                    iz = hi[2] if dz else lo[2]
                    wz = whi[2] if dz else wlo[2]
                    idx_rows.append(((b * d + ix) * d + iy) * d + iz)
                    w_rows.append(wx * wy * wz)
        idx_ref[...] = jnp.concatenate(idx_rows, axis=0)
        w_ref[...] = jnp.concatenate(w_rows, axis=0)


_prep = pl.pallas_call(
    _prep_body,
    out_shape=(
        jax.ShapeDtypeStruct((8, _N), jnp.int32),
        jax.ShapeDtypeStruct((8, _N), jnp.float32),
        jax.ShapeDtypeStruct((8, _N), jnp.int32),
        jax.ShapeDtypeStruct((8, _N), jnp.float32),
    ),
    interpret=_INTERPRET,
)


def _gather_body(t0, t1, i0, i1, out0, out1, idx_v, rows_v, sem):
    wid = lax.axis_index("s") * 2 + lax.axis_index("c")
    for t, ihbm, ohbm in ((t0, i0, out0), (t1, i1, out1)):
        pltpu.sync_copy(ihbm.at[pl.ds(wid * 64, 64)], idx_v)
        for c in range(8):
            copies = [
                pltpu.async_copy(
                    t.at[idx_v.at[8 * c + j]],
                    rows_v.at[pl.ds(j * 128, 128)],
                    sem,
                )
                for j in range(8)
            ]
            for cp in copies:
                cp.wait()
            pltpu.sync_copy(
                rows_v, ohbm.at[pl.ds(wid * _ROWS_PER_W + c * 1024, 1024)])


@functools.lru_cache(maxsize=None)
def _get_gather():
    return pl.kernel(
        _gather_body,
        out_type=(
            jax.ShapeDtypeStruct((8 * _N, 32), jnp.float32),
            jax.ShapeDtypeStruct((8 * _N, 32), jnp.float32),
        ),
        mesh=plsc.VectorSubcoreMesh(core_axis_name="c", subcore_axis_name="s"),
        scratch_types=[
            pltpu.VMEM((64, 128), jnp.int32),
            pltpu.VMEM((1024, 32), jnp.float32),
            pltpu.SemaphoreType.DMA,
        ],
        compiler_params=pltpu.CompilerParams(use_tc_tiling_on_sc=False),
    )


def _combine_body(r0, w0, r1, w1, o_ref):
    outs = []
    for r, w in ((r0, w0), (r1, w1)):
        acc = r[0] * w[0][:, None]
        for k in range(1, 8):
            acc = acc + r[k] * w[k][:, None]
        outs.append(acc)
    o_ref[...] = jnp.concatenate(outs, axis=-1)


_combine = pl.pallas_call(
    _combine_body,
    grid=(_N // _NB,),
    in_specs=[
        pl.BlockSpec((8, _NB, 32), lambda i: (0, i, 0)),
        pl.BlockSpec((8, _NB), lambda i: (0, i)),
        pl.BlockSpec((8, _NB, 32), lambda i: (0, i, 0)),
        pl.BlockSpec((8, _NB), lambda i: (0, i)),
    ],
    out_specs=pl.BlockSpec((_NB, 64), lambda i: (i, 0)),
    out_shape=jax.ShapeDtypeStruct((_N, 64), jnp.float32),
    interpret=_INTERPRET,
)


def kernel(feat0, feat1, mesh_coords):
    B, M, _ = mesh_coords.shape
    coords_t = mesh_coords.reshape(_N, 3).T
    idx0, w0, idx1, w1 = _prep(coords_t)
    rows0, rows1 = _get_gather()(
        feat0.reshape(-1, 32), feat1.reshape(-1, 32),
        idx0.reshape(2048, 128), idx1.reshape(2048, 128))
    out = _combine(rows0.reshape(8, _N, 32), w0, rows1.reshape(8, _N, 32), w1)
    return out.reshape(B, M, 64)
